# baseline (speedup 1.0000x reference)
import dataclasses

import jax
import jax.numpy as jnp
from jax import lax
from jax.experimental import pallas as pl
from jax.experimental.pallas import tpu as pltpu
from jax.experimental.pallas import tpu_sc as plsc

N_USERS = 5000
N_ITEMS = 5000
N_NODES = N_USERS + N_ITEMS
HALF = N_USERS
E = 320000
D = 128

NUM_CORES = 2
NUM_SUBCORES = 16
NUM_WORKERS = NUM_CORES * NUM_SUBCORES
CHUNK = 128
IN_CHUNKS_PER_WORKER = 80
E_PAD = NUM_WORKERS * IN_CHUNKS_PER_WORKER * CHUNK
BUCKET_CAP_CHUNKS = 24
BUCKET_CAP = BUCKET_CAP_CHUNKS * CHUNK
PART_ROWS = 4 * NUM_WORKERS * BUCKET_CAP_CHUNKS
PHASE_CHUNKS = 2 * BUCKET_CAP_CHUNKS
X_ROWS_PER_TILE = 312
PACK_SHIFT = 14
PACK_MASK = (1 << PACK_SHIFT) - 1

_GATHER_DNUMS = lax.GatherDimensionNumbers(
    offset_dims=(), collapsed_slice_dims=(0,), start_index_map=(0,))


def _lane_bcast(vv, d):
    idx = jnp.full((16, 1), d, dtype=jnp.int32)
    return lax.gather(vv, idx, _GATHER_DNUMS, (1,),
                      mode=lax.GatherScatterMode.PROMISE_IN_BOUNDS)


def _lane_bcast_dyn(vv, d):
    idx = jnp.broadcast_to(jnp.reshape(d, (1, 1)), (16, 1)).astype(jnp.int32)
    return lax.gather(vv, idx, _GATHER_DNUMS, (1,),
                      mode=lax.GatherScatterMode.PROMISE_IN_BOUNDS)


def _noop_packed(b):
    return ((b & 1) * HALF << PACK_SHIFT) | ((b >> 1) * HALF)


def _prepass_body(packed_hbm, val_hbm, pk_out_hbm, val_out_hbm,
                  pk_in, val_in, pk_bk, val_bk):
    cid = lax.axis_index("c")
    sid = lax.axis_index("s")
    wid = sid * NUM_CORES + cid

    slab = pl.multiple_of(wid * IN_CHUNKS_PER_WORKER, 8)
    pltpu.sync_copy(packed_hbm.at[pl.ds(slab, IN_CHUNKS_PER_WORKER)], pk_in)
    pltpu.sync_copy(val_hbm.at[pl.ds(slab, IN_CHUNKS_PER_WORKER)], val_in)

    for b in range(4):
        noop = jnp.full((16,), _noop_packed(b), jnp.int32)
        zero = jnp.zeros((16,), jnp.float32)

        @pl.loop(0, BUCKET_CAP // 16)
        def _fill(g):
            pk_bk[pl.ds(b * BUCKET_CAP + g * 16, 16)] = noop
            val_bk[pl.ds(b * BUCKET_CAP + g * 16, 16)] = zero

    def group(carry, r, g):
        sl = pl.ds(g * 16, 16)
        p16 = pk_in[r, sl]
        v16 = val_in[r, sl]
        valid = p16 >= 0
        srcv = jnp.bitwise_and(p16, PACK_MASK)
        dstv = lax.shift_right_logical(
            jnp.bitwise_and(p16, 0x7FFFFFFF), PACK_SHIFT)
        key = (jnp.where(srcv >= HALF, 2, 0) + jnp.where(dstv >= HALF, 1, 0))
        new_carry = []
        for b in range(4):
            cnt = carry[b]
            m = jnp.logical_and(key == b, valid)
            pos = jnp.minimum(cnt, BUCKET_CAP - 16)
            plsc.store_compressed(pk_bk.at[pl.ds(b * BUCKET_CAP + pos, 16)],
                                  p16, mask=m)
            plsc.store_compressed(val_bk.at[pl.ds(b * BUCKET_CAP + pos, 16)],
                                  v16, mask=m)
            npop = jnp.max(plsc.all_reduce_population_count(m))
            new_carry.append(pos + npop)
        return new_carry

    def row(carry, r):
        for g in range(CHUNK // 16):
            carry = group(carry, r, g)
        return carry

    zero_i = jnp.zeros((), jnp.int32)
    lax.fori_loop(0, IN_CHUNKS_PER_WORKER,
                  lambda r, c: row(c, r),
                  [zero_i, zero_i, zero_i, zero_i])

    for b in range(4):
        base = pl.multiple_of((b * NUM_WORKERS + wid) * BUCKET_CAP, 8)
        pltpu.sync_copy(pk_bk.at[pl.ds(b * BUCKET_CAP, BUCKET_CAP)],
                        pk_out_hbm.at[pl.ds(base, BUCKET_CAP)])
        pltpu.sync_copy(val_bk.at[pl.ds(b * BUCKET_CAP, BUCKET_CAP)],
                        val_out_hbm.at[pl.ds(base, BUCKET_CAP)])


def _spmm_body(x_hbm, pk_hbm, valp_hbm, out_hbm,
               pk_v, val_v, srcg_v, dstg_v, rows_v, x_sh, acc_sh, gsem,
               ssem):
    cid = lax.axis_index("c")
    sid = lax.axis_index("s")

    SUB = 96
    NSUB = PHASE_CHUNKS * CHUNK // SUB

    def start_gather(k):
        pltpu.async_copy(x_sh.at[srcg_v.at[k]], rows_v.at[k], gsem.at[k])

    def wait_gather(k):
        pltpu.make_async_copy(x_sh.at[srcg_v.at[k]], rows_v.at[k],
                              gsem.at[k]).wait()

    def start_scatter(k, slot):
        pltpu.async_copy(rows_v.at[k], acc_sh.at[dstg_v.at[slot]],
                         ssem.at[k], add=True)

    def wait_scatter(k, slot):
        pltpu.make_async_copy(rows_v.at[k], acc_sh.at[dstg_v.at[slot]],
                              ssem.at[k]).wait()

    def unpack_idx(t, k, slot, h):
        src_base = cid * HALF
        dst_base = h * HALF
        for g in range(SUB // 16):
            f = t * SUB + g * 16
            row = lax.shift_right_logical(f, 7)
            off = jnp.bitwise_and(f, CHUNK - 1)
            p = pk_v[row, pl.ds(off, 16)]
            so = pl.ds(g * 16, 16)
            srcg_v[k, so] = jnp.bitwise_and(p, PACK_MASK) - src_base
            dstg_v[slot, so] = (lax.shift_right_logical(p, PACK_SHIFT)
                                - dst_base)

    def scale_chunk(t, k):
        buf = rows_v.at[k]

        @pl.loop(0, SUB // 16)
        def _group(g):
            f = t * SUB + g * 16
            row = lax.shift_right_logical(f, 7)
            off = jnp.bitwise_and(f, CHUNK - 1)
            vv = val_v[row, pl.ds(off, 16)]
            base = g * 16
            for d in range(16):
                vb = _lane_bcast(vv, d)
                e = base + d
                for s in range(D // 16):
                    sl = pl.ds(s * 16, 16)
                    buf[e, sl] = buf[e, sl] * vb

    def zero_acc():
        @pl.loop(0, SUB)
        def _zero_rows(r):
            for s in range(D // 16):
                rows_v[0, r, pl.ds(s * 16, 16)] = jnp.zeros((16,),
                                                            jnp.float32)

        arow = sid * X_ROWS_PER_TILE
        for k in range(3):
            pltpu.sync_copy(rows_v.at[0],
                            acc_sh.at[pl.ds(arow + k * SUB, SUB)])
        pltpu.sync_copy(
            rows_v.at[0].at[pl.ds(0, X_ROWS_PER_TILE - 3 * SUB)],
            acc_sh.at[pl.ds(arow + 3 * SUB, X_ROWS_PER_TILE - 3 * SUB)])

        @pl.when(sid == NUM_SUBCORES - 1)
        def _tail():
            pltpu.sync_copy(
                rows_v.at[0].at[pl.ds(0, HALF - NUM_SUBCORES
                                      * X_ROWS_PER_TILE)],
                acc_sh.at[pl.ds(NUM_SUBCORES * X_ROWS_PER_TILE,
                                HALF - NUM_SUBCORES * X_ROWS_PER_TILE)])

    xbase = pl.multiple_of(cid * HALF + sid * X_ROWS_PER_TILE, 8)
    pltpu.sync_copy(x_hbm.at[pl.ds(xbase, X_ROWS_PER_TILE)],
                    x_sh.at[pl.ds(sid * X_ROWS_PER_TILE, X_ROWS_PER_TILE)])

    @pl.when(sid == NUM_SUBCORES - 1)
    def _xtail():
        t = NUM_SUBCORES * X_ROWS_PER_TILE
        pltpu.sync_copy(x_hbm.at[pl.ds(pl.multiple_of(cid * HALF + t, 8),
                                       HALF - t)],
                        x_sh.at[pl.ds(t, HALF - t)])

    for p in range(2):
        h = jnp.bitwise_xor(cid, p)
        bkt = 2 * cid + h

        zero_acc()

        lbase = pl.multiple_of(
            (bkt * NUM_WORKERS + 2 * sid) * BUCKET_CAP_CHUNKS, 8)
        pltpu.sync_copy(pk_hbm.at[pl.ds(lbase, PHASE_CHUNKS)], pk_v)
        pltpu.sync_copy(valp_hbm.at[pl.ds(lbase, PHASE_CHUNKS)], val_v)

        plsc.subcore_barrier()

        zero32 = jnp.zeros((), jnp.int32)
        for t0 in range(2):
            unpack_idx(zero32 + t0, t0, zero32 + t0, h)
            start_gather(t0)

        @pl.loop(0, NSUB)
        def _body(t):
            k = lax.rem(t, 3)
            wait_gather(k)
            scale_chunk(t, k)
            start_scatter(k, lax.rem(t, 6))

            @pl.when(t >= 1)
            def _wsc():
                wait_scatter(lax.rem(t - 1, 3), lax.rem(t - 1, 6))

            @pl.when(t < NSUB - 2)
            def _next():
                k2 = lax.rem(t + 2, 3)
                unpack_idx(t + 2, k2, lax.rem(t + 2, 6), h)
                start_gather(k2)

        wait_scatter((NSUB - 1) % 3, zero32 + ((NSUB - 1) % 6))

        plsc.subcore_barrier()

        orow = pl.multiple_of((2 * p + h) * HALF + sid * X_ROWS_PER_TILE, 8)
        obase = pl.multiple_of(sid * X_ROWS_PER_TILE, 8)
        pltpu.sync_copy(acc_sh.at[pl.ds(obase, X_ROWS_PER_TILE)],
                        out_hbm.at[pl.ds(orow, X_ROWS_PER_TILE)])

        @pl.when(sid == NUM_SUBCORES - 1)
        def _otail():
            t = NUM_SUBCORES * X_ROWS_PER_TILE
            pltpu.sync_copy(
                acc_sh.at[pl.ds(t, HALF - t)],
                out_hbm.at[pl.ds(pl.multiple_of((2 * p + h) * HALF + t, 8),
                                 HALF - t)])

        plsc.subcore_barrier()


_sc_params = pltpu.CompilerParams()
if "needs_layout_passes" in pltpu.CompilerParams.__dataclass_fields__:
    _sc_params = dataclasses.replace(_sc_params, needs_layout_passes=False)

_MESH = plsc.VectorSubcoreMesh(core_axis_name="c", subcore_axis_name="s")

_prepass = pl.kernel(
    _prepass_body,
    out_type=[
        jax.ShapeDtypeStruct((PART_ROWS * CHUNK,), jnp.int32),
        jax.ShapeDtypeStruct((PART_ROWS * CHUNK,), jnp.float32),
    ],
    mesh=_MESH,
    compiler_params=_sc_params,
    scratch_types=[
        pltpu.VMEM((IN_CHUNKS_PER_WORKER, CHUNK), jnp.int32),
        pltpu.VMEM((IN_CHUNKS_PER_WORKER, CHUNK), jnp.float32),
        pltpu.VMEM((4 * BUCKET_CAP,), jnp.int32),
        pltpu.VMEM((4 * BUCKET_CAP,), jnp.float32),
    ],
)

_spmm = pl.kernel(
    _spmm_body,
    out_type=jax.ShapeDtypeStruct((4 * HALF, D), jnp.float32),
    mesh=_MESH,
    compiler_params=_sc_params,
    scratch_types=[
        pltpu.VMEM((PHASE_CHUNKS, CHUNK), jnp.int32),
        pltpu.VMEM((PHASE_CHUNKS, CHUNK), jnp.float32),
        pltpu.VMEM((3, 96), jnp.int32),
        pltpu.VMEM((6, 96), jnp.int32),
        pltpu.VMEM((3, 96, D), jnp.float32),
        pltpu.VMEM_SHARED((HALF, D), jnp.float32),
        pltpu.VMEM_SHARED((HALF, D), jnp.float32),
        pltpu.SemaphoreType.DMA((3,)),
        pltpu.SemaphoreType.DMA((3,)),
    ],
)


ROW_BLOCK = 1000


def _lin_kernel(x_ref, w_ref, b_ref, o_ref):
    o_ref[...] = lax.dot_general(
        x_ref[...], w_ref[...], (((1,), (1,)), ((), ())),
        preferred_element_type=jnp.float32,
    ) + b_ref[...]


def _linear(x, w, b):
    return pl.pallas_call(
        _lin_kernel,
        grid=(N_NODES // ROW_BLOCK,),
        in_specs=[
            pl.BlockSpec((ROW_BLOCK, D), lambda i: (i, 0)),
            pl.BlockSpec((D, D), lambda i: (0, 0)),
            pl.BlockSpec((1, D), lambda i: (0, 0)),
        ],
        out_specs=pl.BlockSpec((ROW_BLOCK, D), lambda i: (i, 0)),
        out_shape=jax.ShapeDtypeStruct((N_NODES, D), jnp.float32),
    )(x, w, b)


def _fuse_kernel(p0_ref, p1_ref, x_ref, w_ref, b_ref, o_ref):
    h = jnp.maximum(p0_ref[...] + p1_ref[...] + x_ref[...], 0.0)
    o_ref[...] = lax.dot_general(
        h, w_ref[...], (((1,), (1,)), ((), ())),
        preferred_element_type=jnp.float32,
    ) + b_ref[...]


def _fused_layer2(p0, p1, x0, w, b):
    return pl.pallas_call(
        _fuse_kernel,
        grid=(N_NODES // ROW_BLOCK,),
        in_specs=[
            pl.BlockSpec((ROW_BLOCK, D), lambda i: (i, 0)),
            pl.BlockSpec((ROW_BLOCK, D), lambda i: (i, 0)),
            pl.BlockSpec((ROW_BLOCK, D), lambda i: (i, 0)),
            pl.BlockSpec((D, D), lambda i: (0, 0)),
            pl.BlockSpec((1, D), lambda i: (0, 0)),
        ],
        out_specs=pl.BlockSpec((ROW_BLOCK, D), lambda i: (i, 0)),
        out_shape=jax.ShapeDtypeStruct((N_NODES, D), jnp.float32),
    )(p0, p1, x0, w, b)


def _add_kernel(q0_ref, q1_ref, o_ref):
    o_ref[...] = q0_ref[...] + q1_ref[...]


def _add_partials(q0, q1):
    return pl.pallas_call(
        _add_kernel,
        grid=(N_NODES // ROW_BLOCK,),
        in_specs=[
            pl.BlockSpec((ROW_BLOCK, D), lambda i: (i, 0)),
            pl.BlockSpec((ROW_BLOCK, D), lambda i: (i, 0)),
        ],
        out_specs=pl.BlockSpec((ROW_BLOCK, D), lambda i: (i, 0)),
        out_shape=jax.ShapeDtypeStruct((N_NODES, D), jnp.float32),
    )(q0, q1)


def kernel(user_feat, item_feat, A_indices, A_values, W1, b1, W2, b2):
    x0 = jnp.concatenate([user_feat, item_feat], axis=0)
    b1r = b1.reshape(1, D)
    b2r = b2.reshape(1, D)

    pad = E_PAD - E
    packed = jnp.bitwise_or(jnp.left_shift(A_indices[0], PACK_SHIFT),
                            A_indices[1])
    packed = jnp.concatenate([packed, jnp.full((pad,), -1, jnp.int32)])
    val = jnp.concatenate([A_values, jnp.zeros((pad,), jnp.float32)])
    rows = NUM_WORKERS * IN_CHUNKS_PER_WORKER
    packed_arr = packed.reshape(rows, CHUNK)
    val_arr = val.reshape(rows, CHUNK)

    pk_flat, val_flat = _prepass(packed_arr, val_arr)
    pk_part = pk_flat.reshape(PART_ROWS, CHUNK)
    val_part = val_flat.reshape(PART_ROWS, CHUNK)

    t1 = _linear(x0, W1, b1r)
    p = _spmm(t1, pk_part, val_part)
    t2 = _fused_layer2(p[:N_NODES], p[N_NODES:], x0, W2, b2r)
    q = _spmm(t2, pk_part, val_part)
    s = _add_partials(q[:N_NODES], q[N_NODES:])
    return s[:N_USERS], s[N_USERS:]


# speedup vs baseline: 2.2237x; 2.2237x over previous
import dataclasses

import jax
import jax.numpy as jnp
from jax import lax
from jax.experimental import pallas as pl
from jax.experimental.pallas import tpu as pltpu
from jax.experimental.pallas import tpu_sc as plsc

N_USERS = 5000
N_ITEMS = 5000
N_NODES = N_USERS + N_ITEMS
HALF = N_USERS
E = 320000
D = 128

NUM_CORES = 2
NUM_SUBCORES = 16
NUM_WORKERS = NUM_CORES * NUM_SUBCORES
CHUNK = 128
IN_CHUNKS_PER_WORKER = 80
E_PAD = NUM_WORKERS * IN_CHUNKS_PER_WORKER * CHUNK
BUCKET_CAP_CHUNKS = 24
BUCKET_CAP = BUCKET_CAP_CHUNKS * CHUNK
PART_ROWS = 4 * NUM_WORKERS * BUCKET_CAP_CHUNKS
PHASE_CHUNKS = 2 * BUCKET_CAP_CHUNKS
X_ROWS_PER_TILE = 312
PACK_SHIFT = 14
PACK_MASK = (1 << PACK_SHIFT) - 1

_GATHER_DNUMS = lax.GatherDimensionNumbers(
    offset_dims=(), collapsed_slice_dims=(0,), start_index_map=(0,))


def _lane_bcast(vv, d):
    idx = jnp.full((16, 1), d, dtype=jnp.int32)
    return lax.gather(vv, idx, _GATHER_DNUMS, (1,),
                      mode=lax.GatherScatterMode.PROMISE_IN_BOUNDS)


def _lane_bcast_dyn(vv, d):
    idx = jnp.broadcast_to(jnp.reshape(d, (1, 1)), (16, 1)).astype(jnp.int32)
    return lax.gather(vv, idx, _GATHER_DNUMS, (1,),
                      mode=lax.GatherScatterMode.PROMISE_IN_BOUNDS)


def _noop_packed(b):
    return ((b & 1) * HALF << PACK_SHIFT) | ((b >> 1) * HALF)


def _prepass_body(packed_hbm, val_hbm, pk_out_hbm, val_out_hbm,
                  pk_in, val_in, pk_bk, val_bk):
    cid = lax.axis_index("c")
    sid = lax.axis_index("s")
    wid = sid * NUM_CORES + cid

    slab = pl.multiple_of(wid * IN_CHUNKS_PER_WORKER, 8)
    pltpu.sync_copy(packed_hbm.at[pl.ds(slab, IN_CHUNKS_PER_WORKER)], pk_in)
    pltpu.sync_copy(val_hbm.at[pl.ds(slab, IN_CHUNKS_PER_WORKER)], val_in)

    for b in range(4):
        noop = jnp.full((16,), _noop_packed(b), jnp.int32)
        zero = jnp.zeros((16,), jnp.float32)

        @pl.loop(0, BUCKET_CAP // 16)
        def _fill(g):
            pk_bk[pl.ds(b * BUCKET_CAP + g * 16, 16)] = noop
            val_bk[pl.ds(b * BUCKET_CAP + g * 16, 16)] = zero

    def group(carry, r, g):
        sl = pl.ds(g * 16, 16)
        p16 = pk_in[r, sl]
        v16 = val_in[r, sl]
        valid = p16 >= 0
        srcv = jnp.bitwise_and(p16, PACK_MASK)
        dstv = lax.shift_right_logical(
            jnp.bitwise_and(p16, 0x7FFFFFFF), PACK_SHIFT)
        key = (jnp.where(srcv >= HALF, 2, 0) + jnp.where(dstv >= HALF, 1, 0))
        new_carry = []
        for b in range(4):
            cnt = carry[b]
            m = jnp.logical_and(key == b, valid)
            pos = jnp.minimum(cnt, BUCKET_CAP - 16)
            plsc.store_compressed(pk_bk.at[pl.ds(b * BUCKET_CAP + pos, 16)],
                                  p16, mask=m)
            plsc.store_compressed(val_bk.at[pl.ds(b * BUCKET_CAP + pos, 16)],
                                  v16, mask=m)
            npop = jnp.max(plsc.all_reduce_population_count(m))
            new_carry.append(pos + npop)
        return new_carry

    def row(carry, r):
        for g in range(CHUNK // 16):
            carry = group(carry, r, g)
        return carry

    zero_i = jnp.zeros((), jnp.int32)
    lax.fori_loop(0, IN_CHUNKS_PER_WORKER,
                  lambda r, c: row(c, r),
                  [zero_i, zero_i, zero_i, zero_i])

    for b in range(4):
        base = pl.multiple_of((b * NUM_WORKERS + wid) * BUCKET_CAP, 8)
        pltpu.sync_copy(pk_bk.at[pl.ds(b * BUCKET_CAP, BUCKET_CAP)],
                        pk_out_hbm.at[pl.ds(base, BUCKET_CAP)])
        pltpu.sync_copy(val_bk.at[pl.ds(b * BUCKET_CAP, BUCKET_CAP)],
                        val_out_hbm.at[pl.ds(base, BUCKET_CAP)])


def _spmm_body(x_hbm, pk_hbm, valp_hbm, out_hbm,
               pk_v, val_v, srcg_v, dstg_v, rows_v, x_sh, acc_sh, gsem,
               ssem):
    cid = lax.axis_index("c")
    sid = lax.axis_index("s")

    SUB = 96
    NSUB = PHASE_CHUNKS * CHUNK // SUB

    def start_gather(k):
        pltpu.async_copy(x_sh.at[srcg_v.at[k]], rows_v.at[k], gsem.at[k])

    def wait_gather(k):
        pltpu.make_async_copy(x_sh.at[srcg_v.at[k]], rows_v.at[k],
                              gsem.at[k]).wait()

    def start_scatter(k, slot):
        pltpu.async_copy(rows_v.at[k], acc_sh.at[dstg_v.at[slot]],
                         ssem.at[k], add=True)

    def wait_scatter(k, slot):
        pltpu.make_async_copy(rows_v.at[k], acc_sh.at[dstg_v.at[slot]],
                              ssem.at[k]).wait()

    def unpack_idx(t, k, slot, h):
        src_base = cid * HALF
        dst_base = h * HALF
        for g in range(SUB // 16):
            f = t * SUB + g * 16
            row = lax.shift_right_logical(f, 7)
            off = jnp.bitwise_and(f, CHUNK - 1)
            p = pk_v[row, pl.ds(off, 16)]
            so = pl.ds(g * 16, 16)
            srcg_v[k, so] = jnp.bitwise_and(p, PACK_MASK) - src_base
            dstg_v[slot, so] = (lax.shift_right_logical(p, PACK_SHIFT)
                                - dst_base)

    def scale_chunk(t, k):
        buf = rows_v.at[k]

        @pl.loop(0, SUB // 16)
        def _group(g):
            f = t * SUB + g * 16
            row = lax.shift_right_logical(f, 7)
            off = jnp.bitwise_and(f, CHUNK - 1)
            vv = val_v[row, pl.ds(off, 16)]
            base = g * 16

            @pl.loop(0, 16, step=4)
            def _quad(d0):
                for dd in range(4):
                    vb = _lane_bcast_dyn(vv, d0 + dd)
                    e = base + d0 + dd
                    for s in range(D // 16):
                        sl = pl.ds(s * 16, 16)
                        buf[e, sl] = buf[e, sl] * vb

    def zero_acc():
        @pl.loop(0, SUB)
        def _zero_rows(r):
            for s in range(D // 16):
                rows_v[0, r, pl.ds(s * 16, 16)] = jnp.zeros((16,),
                                                            jnp.float32)

        arow = sid * X_ROWS_PER_TILE
        for k in range(3):
            pltpu.sync_copy(rows_v.at[0],
                            acc_sh.at[pl.ds(arow + k * SUB, SUB)])
        pltpu.sync_copy(
            rows_v.at[0].at[pl.ds(0, X_ROWS_PER_TILE - 3 * SUB)],
            acc_sh.at[pl.ds(arow + 3 * SUB, X_ROWS_PER_TILE - 3 * SUB)])

        @pl.when(sid == NUM_SUBCORES - 1)
        def _tail():
            pltpu.sync_copy(
                rows_v.at[0].at[pl.ds(0, HALF - NUM_SUBCORES
                                      * X_ROWS_PER_TILE)],
                acc_sh.at[pl.ds(NUM_SUBCORES * X_ROWS_PER_TILE,
                                HALF - NUM_SUBCORES * X_ROWS_PER_TILE)])

    xbase = pl.multiple_of(cid * HALF + sid * X_ROWS_PER_TILE, 8)
    pltpu.sync_copy(x_hbm.at[pl.ds(xbase, X_ROWS_PER_TILE)],
                    x_sh.at[pl.ds(sid * X_ROWS_PER_TILE, X_ROWS_PER_TILE)])

    @pl.when(sid == NUM_SUBCORES - 1)
    def _xtail():
        t = NUM_SUBCORES * X_ROWS_PER_TILE
        pltpu.sync_copy(x_hbm.at[pl.ds(pl.multiple_of(cid * HALF + t, 8),
                                       HALF - t)],
                        x_sh.at[pl.ds(t, HALF - t)])

    for p in range(2):
        h = jnp.bitwise_xor(cid, p)
        bkt = 2 * cid + h

        zero_acc()

        lbase = pl.multiple_of(
            (bkt * NUM_WORKERS + 2 * sid) * BUCKET_CAP_CHUNKS, 8)
        pltpu.sync_copy(pk_hbm.at[pl.ds(lbase, PHASE_CHUNKS)], pk_v)
        pltpu.sync_copy(valp_hbm.at[pl.ds(lbase, PHASE_CHUNKS)], val_v)

        plsc.subcore_barrier()

        zero32 = jnp.zeros((), jnp.int32)
        for t0 in range(2):
            unpack_idx(zero32 + t0, t0, zero32 + t0, h)
            start_gather(t0)

        @pl.loop(0, NSUB)
        def _body(t):
            k = lax.rem(t, 3)
            wait_gather(k)
            scale_chunk(t, k)
            start_scatter(k, lax.rem(t, 6))

            @pl.when(t >= 1)
            def _wsc():
                wait_scatter(lax.rem(t - 1, 3), lax.rem(t - 1, 6))

            @pl.when(t < NSUB - 2)
            def _next():
                k2 = lax.rem(t + 2, 3)
                unpack_idx(t + 2, k2, lax.rem(t + 2, 6), h)
                start_gather(k2)

        wait_scatter((NSUB - 1) % 3, zero32 + ((NSUB - 1) % 6))

        plsc.subcore_barrier()

        orow = pl.multiple_of((2 * p + h) * HALF + sid * X_ROWS_PER_TILE, 8)
        obase = pl.multiple_of(sid * X_ROWS_PER_TILE, 8)
        pltpu.sync_copy(acc_sh.at[pl.ds(obase, X_ROWS_PER_TILE)],
                        out_hbm.at[pl.ds(orow, X_ROWS_PER_TILE)])

        @pl.when(sid == NUM_SUBCORES - 1)
        def _otail():
            t = NUM_SUBCORES * X_ROWS_PER_TILE
            pltpu.sync_copy(
                acc_sh.at[pl.ds(t, HALF - t)],
                out_hbm.at[pl.ds(pl.multiple_of((2 * p + h) * HALF + t, 8),
                                 HALF - t)])

        plsc.subcore_barrier()


_sc_params = pltpu.CompilerParams()
if "needs_layout_passes" in pltpu.CompilerParams.__dataclass_fields__:
    _sc_params = dataclasses.replace(_sc_params, needs_layout_passes=False)

_MESH = plsc.VectorSubcoreMesh(core_axis_name="c", subcore_axis_name="s")

_prepass = pl.kernel(
    _prepass_body,
    out_type=[
        jax.ShapeDtypeStruct((PART_ROWS * CHUNK,), jnp.int32),
        jax.ShapeDtypeStruct((PART_ROWS * CHUNK,), jnp.float32),
    ],
    mesh=_MESH,
    compiler_params=_sc_params,
    scratch_types=[
        pltpu.VMEM((IN_CHUNKS_PER_WORKER, CHUNK), jnp.int32),
        pltpu.VMEM((IN_CHUNKS_PER_WORKER, CHUNK), jnp.float32),
        pltpu.VMEM((4 * BUCKET_CAP,), jnp.int32),
        pltpu.VMEM((4 * BUCKET_CAP,), jnp.float32),
    ],
)

_spmm = pl.kernel(
    _spmm_body,
    out_type=jax.ShapeDtypeStruct((4 * HALF, D), jnp.float32),
    mesh=_MESH,
    compiler_params=_sc_params,
    scratch_types=[
        pltpu.VMEM((PHASE_CHUNKS, CHUNK), jnp.int32),
        pltpu.VMEM((PHASE_CHUNKS, CHUNK), jnp.float32),
        pltpu.VMEM((3, 96), jnp.int32),
        pltpu.VMEM((6, 96), jnp.int32),
        pltpu.VMEM((3, 96, D), jnp.float32),
        pltpu.VMEM_SHARED((HALF, D), jnp.float32),
        pltpu.VMEM_SHARED((HALF, D), jnp.float32),
        pltpu.SemaphoreType.DMA((3,)),
        pltpu.SemaphoreType.DMA((3,)),
    ],
)


ROW_BLOCK = 1000


def _lin_kernel(x_ref, w_ref, b_ref, o_ref):
    o_ref[...] = lax.dot_general(
        x_ref[...], w_ref[...], (((1,), (1,)), ((), ())),
        preferred_element_type=jnp.float32,
    ) + b_ref[...]


def _linear(x, w, b):
    return pl.pallas_call(
        _lin_kernel,
        grid=(N_NODES // ROW_BLOCK,),
        in_specs=[
            pl.BlockSpec((ROW_BLOCK, D), lambda i: (i, 0)),
            pl.BlockSpec((D, D), lambda i: (0, 0)),
            pl.BlockSpec((1, D), lambda i: (0, 0)),
        ],
        out_specs=pl.BlockSpec((ROW_BLOCK, D), lambda i: (i, 0)),
        out_shape=jax.ShapeDtypeStruct((N_NODES, D), jnp.float32),
    )(x, w, b)


def _fuse_kernel(p0_ref, p1_ref, x_ref, w_ref, b_ref, o_ref):
    h = jnp.maximum(p0_ref[...] + p1_ref[...] + x_ref[...], 0.0)
    o_ref[...] = lax.dot_general(
        h, w_ref[...], (((1,), (1,)), ((), ())),
        preferred_element_type=jnp.float32,
    ) + b_ref[...]


def _fused_layer2(p0, p1, x0, w, b):
    return pl.pallas_call(
        _fuse_kernel,
        grid=(N_NODES // ROW_BLOCK,),
        in_specs=[
            pl.BlockSpec((ROW_BLOCK, D), lambda i: (i, 0)),
            pl.BlockSpec((ROW_BLOCK, D), lambda i: (i, 0)),
            pl.BlockSpec((ROW_BLOCK, D), lambda i: (i, 0)),
            pl.BlockSpec((D, D), lambda i: (0, 0)),
            pl.BlockSpec((1, D), lambda i: (0, 0)),
        ],
        out_specs=pl.BlockSpec((ROW_BLOCK, D), lambda i: (i, 0)),
        out_shape=jax.ShapeDtypeStruct((N_NODES, D), jnp.float32),
    )(p0, p1, x0, w, b)


def _add_kernel(q0_ref, q1_ref, o_ref):
    o_ref[...] = q0_ref[...] + q1_ref[...]


def _add_partials(q0, q1):
    return pl.pallas_call(
        _add_kernel,
        grid=(N_NODES // ROW_BLOCK,),
        in_specs=[
            pl.BlockSpec((ROW_BLOCK, D), lambda i: (i, 0)),
            pl.BlockSpec((ROW_BLOCK, D), lambda i: (i, 0)),
        ],
        out_specs=pl.BlockSpec((ROW_BLOCK, D), lambda i: (i, 0)),
        out_shape=jax.ShapeDtypeStruct((N_NODES, D), jnp.float32),
    )(q0, q1)


def kernel(user_feat, item_feat, A_indices, A_values, W1, b1, W2, b2):
    x0 = jnp.concatenate([user_feat, item_feat], axis=0)
    b1r = b1.reshape(1, D)
    b2r = b2.reshape(1, D)

    pad = E_PAD - E
    packed = jnp.bitwise_or(jnp.left_shift(A_indices[0], PACK_SHIFT),
                            A_indices[1])
    packed = jnp.concatenate([packed, jnp.full((pad,), -1, jnp.int32)])
    val = jnp.concatenate([A_values, jnp.zeros((pad,), jnp.float32)])
    rows = NUM_WORKERS * IN_CHUNKS_PER_WORKER
    packed_arr = packed.reshape(rows, CHUNK)
    val_arr = val.reshape(rows, CHUNK)

    pk_flat, val_flat = _prepass(packed_arr, val_arr)
    pk_part = pk_flat.reshape(PART_ROWS, CHUNK)
    val_part = val_flat.reshape(PART_ROWS, CHUNK)

    t1 = _linear(x0, W1, b1r)
    p = _spmm(t1, pk_part, val_part)
    t2 = _fused_layer2(p[:N_NODES], p[N_NODES:], x0, W2, b2r)
    q = _spmm(t2, pk_part, val_part)
    s = _add_partials(q[:N_NODES], q[N_NODES:])
    return s[:N_USERS], s[N_USERS:]


# speedup vs baseline: 2.2546x; 1.0139x over previous
import dataclasses

import jax
import jax.numpy as jnp
from jax import lax
from jax.experimental import pallas as pl
from jax.experimental.pallas import tpu as pltpu
from jax.experimental.pallas import tpu_sc as plsc

N_USERS = 5000
N_ITEMS = 5000
N_NODES = N_USERS + N_ITEMS
HALF = N_USERS
E = 320000
D = 128

NUM_CORES = 2
NUM_SUBCORES = 16
NUM_WORKERS = NUM_CORES * NUM_SUBCORES
CHUNK = 128
IN_CHUNKS_PER_WORKER = 80
E_PAD = NUM_WORKERS * IN_CHUNKS_PER_WORKER * CHUNK
BUCKET_CAP_CHUNKS = 24
BUCKET_CAP = BUCKET_CAP_CHUNKS * CHUNK
PART_ROWS = 4 * NUM_WORKERS * BUCKET_CAP_CHUNKS
PHASE_CHUNKS = 2 * BUCKET_CAP_CHUNKS
X_ROWS_PER_TILE = 312
PACK_SHIFT = 14
PACK_MASK = (1 << PACK_SHIFT) - 1

_GATHER_DNUMS = lax.GatherDimensionNumbers(
    offset_dims=(), collapsed_slice_dims=(0,), start_index_map=(0,))


def _lane_bcast(vv, d):
    idx = jnp.full((16, 1), d, dtype=jnp.int32)
    return lax.gather(vv, idx, _GATHER_DNUMS, (1,),
                      mode=lax.GatherScatterMode.PROMISE_IN_BOUNDS)


def _lane_bcast_dyn(vv, d):
    idx = jnp.broadcast_to(jnp.reshape(d, (1, 1)), (16, 1)).astype(jnp.int32)
    return lax.gather(vv, idx, _GATHER_DNUMS, (1,),
                      mode=lax.GatherScatterMode.PROMISE_IN_BOUNDS)


def _noop_packed(b):
    return ((b & 1) * HALF << PACK_SHIFT) | ((b >> 1) * HALF)


def _prepass_body(packed_hbm, val_hbm, pk_out_hbm, val_out_hbm,
                  pk_in, val_in, pk_bk, val_bk):
    cid = lax.axis_index("c")
    sid = lax.axis_index("s")
    wid = sid * NUM_CORES + cid

    slab = pl.multiple_of(wid * IN_CHUNKS_PER_WORKER, 8)
    pltpu.sync_copy(packed_hbm.at[pl.ds(slab, IN_CHUNKS_PER_WORKER)], pk_in)
    pltpu.sync_copy(val_hbm.at[pl.ds(slab, IN_CHUNKS_PER_WORKER)], val_in)

    for b in range(4):
        noop = jnp.full((16,), _noop_packed(b), jnp.int32)
        zero = jnp.zeros((16,), jnp.float32)

        @pl.loop(0, BUCKET_CAP // 16)
        def _fill(g):
            pk_bk[pl.ds(b * BUCKET_CAP + g * 16, 16)] = noop
            val_bk[pl.ds(b * BUCKET_CAP + g * 16, 16)] = zero

    def group(carry, r, g):
        sl = pl.ds(g * 16, 16)
        p16 = pk_in[r, sl]
        v16 = val_in[r, sl]
        valid = p16 >= 0
        srcv = jnp.bitwise_and(p16, PACK_MASK)
        dstv = lax.shift_right_logical(
            jnp.bitwise_and(p16, 0x7FFFFFFF), PACK_SHIFT)
        key = (jnp.where(srcv >= HALF, 2, 0) + jnp.where(dstv >= HALF, 1, 0))
        new_carry = []
        for b in range(4):
            cnt = carry[b]
            m = jnp.logical_and(key == b, valid)
            pos = jnp.minimum(cnt, BUCKET_CAP - 16)
            plsc.store_compressed(pk_bk.at[pl.ds(b * BUCKET_CAP + pos, 16)],
                                  p16, mask=m)
            plsc.store_compressed(val_bk.at[pl.ds(b * BUCKET_CAP + pos, 16)],
                                  v16, mask=m)
            npop = jnp.max(plsc.all_reduce_population_count(m))
            new_carry.append(pos + npop)
        return new_carry

    def row(carry, r):
        for g in range(CHUNK // 16):
            carry = group(carry, r, g)
        return carry

    zero_i = jnp.zeros((), jnp.int32)
    lax.fori_loop(0, IN_CHUNKS_PER_WORKER,
                  lambda r, c: row(c, r),
                  [zero_i, zero_i, zero_i, zero_i])

    for b in range(4):
        base = pl.multiple_of((b * NUM_WORKERS + wid) * BUCKET_CAP, 8)
        pltpu.sync_copy(pk_bk.at[pl.ds(b * BUCKET_CAP, BUCKET_CAP)],
                        pk_out_hbm.at[pl.ds(base, BUCKET_CAP)])
        pltpu.sync_copy(val_bk.at[pl.ds(b * BUCKET_CAP, BUCKET_CAP)],
                        val_out_hbm.at[pl.ds(base, BUCKET_CAP)])


def _spmm_body(x_hbm, pk_hbm, valp_hbm, out_hbm,
               pk_v, val_v, srcg_v, dstg_v, rows_v, x_sh, acc_sh, gsem,
               ssem):
    cid = lax.axis_index("c")
    sid = lax.axis_index("s")

    SUB = 96
    NSUB = PHASE_CHUNKS * CHUNK // SUB

    def start_gather(k):
        pltpu.async_copy(x_sh.at[srcg_v.at[k]], rows_v.at[k], gsem.at[k])

    def wait_gather(k):
        pltpu.make_async_copy(x_sh.at[srcg_v.at[k]], rows_v.at[k],
                              gsem.at[k]).wait()

    def start_scatter(k, slot):
        pltpu.async_copy(rows_v.at[k], acc_sh.at[dstg_v.at[slot]],
                         ssem.at[k], add=True)

    def wait_scatter(k, slot):
        pltpu.make_async_copy(rows_v.at[k], acc_sh.at[dstg_v.at[slot]],
                              ssem.at[k]).wait()

    def unpack_idx(t, k, slot, h):
        src_base = cid * HALF
        dst_base = h * HALF
        for g in range(SUB // 16):
            f = t * SUB + g * 16
            row = lax.shift_right_logical(f, 7)
            off = jnp.bitwise_and(f, CHUNK - 1)
            p = pk_v[row, pl.ds(off, 16)]
            so = pl.ds(g * 16, 16)
            srcg_v[k, so] = jnp.bitwise_and(p, PACK_MASK) - src_base
            dstg_v[slot, so] = (lax.shift_right_logical(p, PACK_SHIFT)
                                - dst_base)

    def scale_chunk(t, k):
        buf = rows_v.at[k]

        @pl.loop(0, SUB // 16)
        def _group(g):
            f = t * SUB + g * 16
            row = lax.shift_right_logical(f, 7)
            off = jnp.bitwise_and(f, CHUNK - 1)
            vv = val_v[row, pl.ds(off, 16)]
            base = g * 16

            @pl.loop(0, 16, step=4)
            def _quad(d0):
                for dd in range(4):
                    vb = _lane_bcast_dyn(vv, d0 + dd)
                    e = base + d0 + dd
                    for s in range(D // 16):
                        sl = pl.ds(s * 16, 16)
                        buf[e, sl] = buf[e, sl] * vb

    def zero_acc():
        @pl.loop(0, SUB)
        def _zero_rows(r):
            for s in range(D // 16):
                rows_v[0, r, pl.ds(s * 16, 16)] = jnp.zeros((16,),
                                                            jnp.float32)

        arow = sid * X_ROWS_PER_TILE
        for k in range(3):
            pltpu.sync_copy(rows_v.at[0],
                            acc_sh.at[pl.ds(arow + k * SUB, SUB)])
        pltpu.sync_copy(
            rows_v.at[0].at[pl.ds(0, X_ROWS_PER_TILE - 3 * SUB)],
            acc_sh.at[pl.ds(arow + 3 * SUB, X_ROWS_PER_TILE - 3 * SUB)])

        @pl.when(sid == NUM_SUBCORES - 1)
        def _tail():
            pltpu.sync_copy(
                rows_v.at[0].at[pl.ds(0, HALF - NUM_SUBCORES
                                      * X_ROWS_PER_TILE)],
                acc_sh.at[pl.ds(NUM_SUBCORES * X_ROWS_PER_TILE,
                                HALF - NUM_SUBCORES * X_ROWS_PER_TILE)])

    xbase = pl.multiple_of(cid * HALF + sid * X_ROWS_PER_TILE, 8)
    pltpu.sync_copy(x_hbm.at[pl.ds(xbase, X_ROWS_PER_TILE)],
                    x_sh.at[pl.ds(sid * X_ROWS_PER_TILE, X_ROWS_PER_TILE)])

    @pl.when(sid == NUM_SUBCORES - 1)
    def _xtail():
        t = NUM_SUBCORES * X_ROWS_PER_TILE
        pltpu.sync_copy(x_hbm.at[pl.ds(pl.multiple_of(cid * HALF + t, 8),
                                       HALF - t)],
                        x_sh.at[pl.ds(t, HALF - t)])

    for p in range(2):
        h = jnp.bitwise_xor(cid, p)
        bkt = 2 * cid + h

        zero_acc()

        lbase = pl.multiple_of(
            (bkt * NUM_WORKERS + 2 * sid) * BUCKET_CAP_CHUNKS, 8)
        pltpu.sync_copy(pk_hbm.at[pl.ds(lbase, PHASE_CHUNKS)], pk_v)
        pltpu.sync_copy(valp_hbm.at[pl.ds(lbase, PHASE_CHUNKS)], val_v)

        plsc.subcore_barrier()

        zero32 = jnp.zeros((), jnp.int32)
        for t0 in range(2):
            unpack_idx(zero32 + t0, t0, zero32 + t0, h)
            start_gather(t0)

        @pl.loop(0, NSUB)
        def _body(t):
            k = lax.rem(t, 3)
            wait_gather(k)
            scale_chunk(t, k)
            start_scatter(k, lax.rem(t, 6))

            @pl.when(t >= 1)
            def _wsc():
                wait_scatter(lax.rem(t - 1, 3), lax.rem(t - 1, 6))

            @pl.when(t < NSUB - 2)
            def _next():
                k2 = lax.rem(t + 2, 3)
                unpack_idx(t + 2, k2, lax.rem(t + 2, 6), h)
                start_gather(k2)

        wait_scatter((NSUB - 1) % 3, zero32 + ((NSUB - 1) % 6))

        plsc.subcore_barrier()

        orow = pl.multiple_of((2 * p + h) * HALF + sid * X_ROWS_PER_TILE, 8)
        obase = pl.multiple_of(sid * X_ROWS_PER_TILE, 8)
        pltpu.sync_copy(acc_sh.at[pl.ds(obase, X_ROWS_PER_TILE)],
                        out_hbm.at[pl.ds(orow, X_ROWS_PER_TILE)])

        @pl.when(sid == NUM_SUBCORES - 1)
        def _otail():
            t = NUM_SUBCORES * X_ROWS_PER_TILE
            pltpu.sync_copy(
                acc_sh.at[pl.ds(t, HALF - t)],
                out_hbm.at[pl.ds(pl.multiple_of((2 * p + h) * HALF + t, 8),
                                 HALF - t)])

        plsc.subcore_barrier()


_sc_params = pltpu.CompilerParams()
if "needs_layout_passes" in pltpu.CompilerParams.__dataclass_fields__:
    _sc_params = dataclasses.replace(_sc_params, needs_layout_passes=False)

_MESH = plsc.VectorSubcoreMesh(core_axis_name="c", subcore_axis_name="s")

_prepass = pl.kernel(
    _prepass_body,
    out_type=[
        jax.ShapeDtypeStruct((PART_ROWS * CHUNK,), jnp.int32),
        jax.ShapeDtypeStruct((PART_ROWS * CHUNK,), jnp.float32),
    ],
    mesh=_MESH,
    compiler_params=_sc_params,
    scratch_types=[
        pltpu.VMEM((IN_CHUNKS_PER_WORKER, CHUNK), jnp.int32),
        pltpu.VMEM((IN_CHUNKS_PER_WORKER, CHUNK), jnp.float32),
        pltpu.VMEM((4 * BUCKET_CAP,), jnp.int32),
        pltpu.VMEM((4 * BUCKET_CAP,), jnp.float32),
    ],
)

_spmm = pl.kernel(
    _spmm_body,
    out_type=jax.ShapeDtypeStruct((4 * HALF, D), jnp.float32),
    mesh=_MESH,
    compiler_params=_sc_params,
    scratch_types=[
        pltpu.VMEM((PHASE_CHUNKS, CHUNK), jnp.int32),
        pltpu.VMEM((PHASE_CHUNKS, CHUNK), jnp.float32),
        pltpu.VMEM((3, 96), jnp.int32),
        pltpu.VMEM((6, 96), jnp.int32),
        pltpu.VMEM((3, 96, D), jnp.float32),
        pltpu.VMEM_SHARED((HALF, D), jnp.float32),
        pltpu.VMEM_SHARED((HALF, D), jnp.float32),
        pltpu.SemaphoreType.DMA((3,)),
        pltpu.SemaphoreType.DMA((3,)),
    ],
)


ROW_BLOCK = 2000


def _lin_kernel(x_ref, w_ref, b_ref, o_ref):
    o_ref[...] = lax.dot_general(
        x_ref[...], w_ref[...], (((1,), (1,)), ((), ())),
        preferred_element_type=jnp.float32,
    ) + b_ref[...]


def _linear(x, w, b):
    return pl.pallas_call(
        _lin_kernel,
        grid=(N_NODES // ROW_BLOCK,),
        in_specs=[
            pl.BlockSpec((ROW_BLOCK, D), lambda i: (i, 0)),
            pl.BlockSpec((D, D), lambda i: (0, 0)),
            pl.BlockSpec((1, D), lambda i: (0, 0)),
        ],
        out_specs=pl.BlockSpec((ROW_BLOCK, D), lambda i: (i, 0)),
        out_shape=jax.ShapeDtypeStruct((N_NODES, D), jnp.float32),
    )(x, w, b)


def _fuse_kernel(p0_ref, p1_ref, x_ref, w_ref, b_ref, o_ref):
    h = jnp.maximum(p0_ref[...] + p1_ref[...] + x_ref[...], 0.0)
    o_ref[...] = lax.dot_general(
        h, w_ref[...], (((1,), (1,)), ((), ())),
        preferred_element_type=jnp.float32,
    ) + b_ref[...]


def _fused_layer2(p0, p1, x0, w, b):
    return pl.pallas_call(
        _fuse_kernel,
        grid=(N_NODES // ROW_BLOCK,),
        in_specs=[
            pl.BlockSpec((ROW_BLOCK, D), lambda i: (i, 0)),
            pl.BlockSpec((ROW_BLOCK, D), lambda i: (i, 0)),
            pl.BlockSpec((ROW_BLOCK, D), lambda i: (i, 0)),
            pl.BlockSpec((D, D), lambda i: (0, 0)),
            pl.BlockSpec((1, D), lambda i: (0, 0)),
        ],
        out_specs=pl.BlockSpec((ROW_BLOCK, D), lambda i: (i, 0)),
        out_shape=jax.ShapeDtypeStruct((N_NODES, D), jnp.float32),
    )(p0, p1, x0, w, b)


def _add_kernel(q0_ref, q1_ref, o_ref):
    o_ref[...] = q0_ref[...] + q1_ref[...]


def _add_partials(q0, q1):
    return pl.pallas_call(
        _add_kernel,
        grid=(N_NODES // ROW_BLOCK,),
        in_specs=[
            pl.BlockSpec((ROW_BLOCK, D), lambda i: (i, 0)),
            pl.BlockSpec((ROW_BLOCK, D), lambda i: (i, 0)),
        ],
        out_specs=pl.BlockSpec((ROW_BLOCK, D), lambda i: (i, 0)),
        out_shape=jax.ShapeDtypeStruct((N_NODES, D), jnp.float32),
    )(q0, q1)


def kernel(user_feat, item_feat, A_indices, A_values, W1, b1, W2, b2):
    x0 = jnp.concatenate([user_feat, item_feat], axis=0)
    b1r = b1.reshape(1, D)
    b2r = b2.reshape(1, D)

    pad = E_PAD - E
    packed = jnp.bitwise_or(jnp.left_shift(A_indices[0], PACK_SHIFT),
                            A_indices[1])
    packed = jnp.concatenate([packed, jnp.full((pad,), -1, jnp.int32)])
    val = jnp.concatenate([A_values, jnp.zeros((pad,), jnp.float32)])
    rows = NUM_WORKERS * IN_CHUNKS_PER_WORKER
    packed_arr = packed.reshape(rows, CHUNK)
    val_arr = val.reshape(rows, CHUNK)

    pk_flat, val_flat = _prepass(packed_arr, val_arr)
    pk_part = pk_flat.reshape(PART_ROWS, CHUNK)
    val_part = val_flat.reshape(PART_ROWS, CHUNK)

    t1 = _linear(x0, W1, b1r)
    p = _spmm(t1, pk_part, val_part)
    t2 = _fused_layer2(p[:N_NODES], p[N_NODES:], x0, W2, b2r)
    q = _spmm(t2, pk_part, val_part)
    s = _add_partials(q[:N_NODES], q[N_NODES:])
    return s[:N_USERS], s[N_USERS:]
